# baseline (speedup 1.0000x reference)
import functools

import jax
import jax.numpy as jnp
from jax import lax
from jax.experimental import pallas as pl
from jax.experimental.pallas import tpu as pltpu
from jax.experimental.pallas import tpu_sc as plsc

VOCAB = 1000
D = 64
BATCH = 4096
SEQ = 200
VPAD = 1024

NC = 2
NS = 16
NW = NC * NS
IDX_PER_W = SEQ * 128


NBUF = 4


def _body(tabf_hbm, idx_hbm, out_hbm, tab_v, idx_v, buf0, buf1, buf2, buf3,
          sem0, sem1, sem2, sem3):
    wid = lax.axis_index("s") * NC + lax.axis_index("c")
    pltpu.sync_copy(tabf_hbm, tab_v)
    pltpu.sync_copy(idx_hbm.at[pl.ds(wid * IDX_PER_W, IDX_PER_W)], idx_v)

    bufs = (buf0, buf1, buf2, buf3)
    sems = (sem0, sem1, sem2, sem3)
    col0 = wid * 128

    def fill(s, buf):
        for j in range(8):
            idxv = idx_v[pl.ds(s * 128 + j * 16, 16)]

            @plsc.parallel_loop(0, D, unroll=D)
            def _(d):
                col = plsc.load_gather(tab_v, [idxv + d * VPAD])
                buf[d, pl.ds(j * 16, 16)] = col

    def start_out(s, b):
        pltpu.async_copy(bufs[b], out_hbm.at[s, :, pl.ds(col0, 128)], sems[b])

    def wait_out(s, b):
        pltpu.make_async_copy(
            bufs[b], out_hbm.at[s, :, pl.ds(col0, 128)], sems[b]
        ).wait()

    for b in range(NBUF):
        fill(b, bufs[b])
        start_out(b, b)

    def step(i, _):
        for b in range(NBUF):
            s = NBUF + NBUF * i + b
            wait_out(s - NBUF, b)
            fill(s, bufs[b])
            start_out(s, b)
        return 0

    lax.fori_loop(0, (SEQ - NBUF) // NBUF, step, 0, unroll=False)
    for b in range(NBUF):
        wait_out(SEQ - NBUF + b, b)


def _lookup(tabf, idxf):
    mesh = plsc.VectorSubcoreMesh(core_axis_name="c", subcore_axis_name="s")
    f = functools.partial(
        pl.kernel,
        mesh=mesh,
        out_type=jax.ShapeDtypeStruct((SEQ, D, BATCH), jnp.float32),
        scratch_types=[
            pltpu.VMEM((D * VPAD,), jnp.float32),
            pltpu.VMEM((IDX_PER_W,), jnp.int32),
            pltpu.VMEM((D, 128), jnp.float32),
            pltpu.VMEM((D, 128), jnp.float32),
            pltpu.VMEM((D, 128), jnp.float32),
            pltpu.VMEM((D, 128), jnp.float32),
            pltpu.SemaphoreType.DMA,
            pltpu.SemaphoreType.DMA,
            pltpu.SemaphoreType.DMA,
            pltpu.SemaphoreType.DMA,
        ],
        compiler_params=pltpu.CompilerParams(
            use_tc_tiling_on_sc=True, needs_layout_passes=False
        ),
    )(_body)
    return f(tabf, idxf)


@jax.jit
def kernel(word_sequences, embedding_table):
    tabf = (
        jnp.zeros((D, VPAD), jnp.float32)
        .at[:, :VOCAB]
        .set(embedding_table.astype(jnp.float32).T)
        .reshape(-1)
    )
    idxf = (
        word_sequences.astype(jnp.int32)
        .reshape(NW, 128, SEQ)
        .transpose(0, 2, 1)
        .reshape(-1)
    )
    out = _lookup(tabf, idxf)
    return out.transpose(2, 0, 1)


# speedup vs baseline: 1.3704x; 1.3704x over previous
import functools

import jax
import jax.numpy as jnp
from jax import lax
from jax.experimental import pallas as pl
from jax.experimental.pallas import tpu as pltpu
from jax.experimental.pallas import tpu_sc as plsc

VOCAB = 1000
D = 64
BATCH = 4096
SEQ = 200
VPAD = 1024

NC = 2
NS = 16
NW = NC * NS
IDX_PER_W = SEQ * 128


def _body(tabf_hbm, idx_hbm, out_hbm, tab_v, idx_v, buf0, buf1, sem0, sem1):
    wid = lax.axis_index("s") * NC + lax.axis_index("c")
    pltpu.sync_copy(tabf_hbm, tab_v)
    pltpu.sync_copy(idx_hbm.at[pl.ds(wid * IDX_PER_W, IDX_PER_W)], idx_v)

    bufs = (buf0, buf1)
    sems = (sem0, sem1)
    col0 = wid * 128

    def fill(s, buf):
        for j in range(8):
            idxv = idx_v[pl.ds(s * 128 + j * 16, 16)]

            @plsc.parallel_loop(0, D, unroll=16)
            def _(d):
                col = plsc.load_gather(tab_v, [idxv + d * VPAD])
                buf[d, pl.ds(j * 16, 16)] = col

    def start_out(s, b):
        pltpu.async_copy(bufs[b], out_hbm.at[s, :, pl.ds(col0, 128)], sems[b])

    def wait_out(s, b):
        pltpu.make_async_copy(
            bufs[b], out_hbm.at[s, :, pl.ds(col0, 128)], sems[b]
        ).wait()

    fill(0, bufs[0])
    start_out(0, 0)
    fill(1, bufs[1])
    start_out(1, 1)

    def step(i, _):
        for b in range(2):
            s = 2 + 2 * i + b
            wait_out(s - 2, b)
            fill(s, bufs[b])
            start_out(s, b)
        return 0

    lax.fori_loop(0, (SEQ - 2) // 2, step, 0, unroll=False)
    wait_out(SEQ - 2, 0)
    wait_out(SEQ - 1, 1)


def _lookup(tabf, idxf):
    mesh = plsc.VectorSubcoreMesh(core_axis_name="c", subcore_axis_name="s")
    f = functools.partial(
        pl.kernel,
        mesh=mesh,
        out_type=jax.ShapeDtypeStruct((SEQ, D, BATCH), jnp.float32),
        scratch_types=[
            pltpu.VMEM((D * VPAD,), jnp.float32),
            pltpu.VMEM((IDX_PER_W,), jnp.int32),
            pltpu.VMEM((D, 128), jnp.float32),
            pltpu.VMEM((D, 128), jnp.float32),
            pltpu.SemaphoreType.DMA,
            pltpu.SemaphoreType.DMA,
        ],
        compiler_params=pltpu.CompilerParams(
            use_tc_tiling_on_sc=True, needs_layout_passes=False
        ),
    )(_body)
    return f(tabf, idxf)


@jax.jit
def kernel(word_sequences, embedding_table):
    tabf = (
        jnp.zeros((D, VPAD), jnp.float32)
        .at[:, :VOCAB]
        .set(embedding_table.astype(jnp.float32).T)
        .reshape(-1)
    )
    idxf = (
        word_sequences.astype(jnp.int32)
        .reshape(NW, 128, SEQ)
        .transpose(0, 2, 1)
        .reshape(-1)
    )
    out = _lookup(tabf, idxf)
    return out.transpose(2, 0, 1)


# speedup vs baseline: 1.3950x; 1.0179x over previous
import functools

import jax
import jax.numpy as jnp
from jax import lax
from jax.experimental import pallas as pl
from jax.experimental.pallas import tpu as pltpu
from jax.experimental.pallas import tpu_sc as plsc

VOCAB = 1000
D = 64
BATCH = 4096
SEQ = 200
VPAD = 1024

NC = 2
NS = 16
NW = NC * NS
DG = 16
BG = 512
NJ = BG // 16
CSTRIDE = DG * VPAD + 16
TABW = 4 * CSTRIDE


def _body(tab_hbm, idx_hbm, out_hbm, tab_v, idx0, idx1, buf0, buf1,
          osem0, osem1, isem0, isem1):
    wid = lax.axis_index("s") * NC + lax.axis_index("c")
    dgrp = lax.rem(wid, 4)
    bgrp = wid // 4
    d0 = dgrp * DG
    b0 = bgrp * BG

    pltpu.sync_copy(tab_hbm.at[pl.ds(dgrp * TABW, TABW)], tab_v)
    bvec = lax.rem(lax.iota(jnp.int32, 16), 4) * (CSTRIDE + 1)

    rings = (idx0, idx1)
    isems = (isem0, isem1)
    bufs = (buf0, buf1)
    osems = (osem0, osem1)

    def idx_src(s):
        return idx_hbm.at[pl.ds(bgrp * (SEQ * BG) + s * BG, BG)]

    def start_idx(s, r):
        pltpu.async_copy(idx_src(s), rings[r], isems[r])

    def wait_idx(s, r):
        pltpu.make_async_copy(idx_src(s), rings[r], isems[r]).wait()

    def fill(buf, ring):
        for j in range(NJ):
            idxv = ring[pl.ds(j * 16, 16)] + bvec

            @plsc.parallel_loop(0, DG, unroll=16)
            def _(d):
                col = plsc.load_gather(tab_v, [idxv + d * VPAD])
                buf[d, pl.ds(j * 16, 16)] = col

    def out_dst(s):
        return out_hbm.at[s, pl.ds(d0, DG), pl.ds(b0, BG)]

    def start_out(s, b):
        pltpu.async_copy(bufs[b], out_dst(s), osems[b])

    def wait_out(s, b):
        pltpu.make_async_copy(bufs[b], out_dst(s), osems[b]).wait()

    start_idx(0, 0)
    start_idx(1, 1)
    for s in range(2):
        wait_idx(s, s)
        fill(bufs[s], rings[s])
        start_idx(s + 2, s)
        start_out(s, s)

    def step(i, _):
        for b in range(2):
            s = 2 + 2 * i + b
            wait_out(s - 2, b)
            wait_idx(s, b)
            fill(bufs[b], rings[b])

            @pl.when(s + 2 < SEQ)
            def _():
                start_idx(s + 2, b)

            start_out(s, b)
        return 0

    lax.fori_loop(0, (SEQ - 2) // 2, step, 0, unroll=False)
    wait_out(SEQ - 2, 0)
    wait_out(SEQ - 1, 1)


def _lookup(tab4, idxf):
    mesh = plsc.VectorSubcoreMesh(core_axis_name="c", subcore_axis_name="s")
    f = functools.partial(
        pl.kernel,
        mesh=mesh,
        out_type=jax.ShapeDtypeStruct((SEQ, D, BATCH), jnp.float32),
        scratch_types=[
            pltpu.VMEM((TABW,), jnp.float32),
            pltpu.VMEM((BG,), jnp.int32),
            pltpu.VMEM((BG,), jnp.int32),
            pltpu.VMEM((DG, BG), jnp.float32),
            pltpu.VMEM((DG, BG), jnp.float32),
            pltpu.SemaphoreType.DMA,
            pltpu.SemaphoreType.DMA,
            pltpu.SemaphoreType.DMA,
            pltpu.SemaphoreType.DMA,
        ],
        compiler_params=pltpu.CompilerParams(
            use_tc_tiling_on_sc=True, needs_layout_passes=False
        ),
    )(_body)
    return f(tab4, idxf)


@jax.jit
def kernel(word_sequences, embedding_table):
    tabT = (
        jnp.zeros((D, VPAD), jnp.float32)
        .at[:, :VOCAB]
        .set(embedding_table.astype(jnp.float32).T)
    )
    slices = tabT.reshape(4, DG * VPAD)
    tab4 = jnp.stack(
        [jnp.pad(slices, ((0, 0), (c, 16 - c))) for c in range(4)],
        axis=1,
    ).reshape(-1)
    idxf = (
        word_sequences.astype(jnp.int32)
        .reshape(8, BG, SEQ)
        .transpose(0, 2, 1)
        .reshape(-1)
    )
    out = _lookup(tab4, idxf)
    return out.transpose(2, 0, 1)
